# baseline (speedup 1.0000x reference)
import functools
import math

import jax
import jax.numpy as jnp
from jax import lax
from jax.experimental import pallas as pl
from jax.experimental.pallas import tpu as pltpu
from jax.experimental.pallas import tpu_sc as plsc

D_MODEL = 64
SCALE = math.sqrt(D_MODEL)
BLK = 128
CHUNK = 256
BPW = 4
VCHUNK = 128
SKEW = 129


def _nw():
    info = plsc.get_sparse_core_info()
    return info.num_cores * info.num_subcores


def _repack_body(lutt_hbm, tail_hbm, out_hbm, in0, in1, ot0, ot1, tin,
                 isem0, isem1, osem0, osem1, *, vocab):
    nc = plsc.get_sparse_core_info().num_cores
    nw = _nw()
    wid = lax.axis_index("s") * nc + lax.axis_index("c")
    nchunks = vocab // VCHUNK
    nmain = nchunks // nw
    nextra = nchunks - nmain * nw
    ins = (in0, in1)
    ots = (ot0, ot1)
    isems = (isem0, isem1)
    osems = (osem0, osem1)

    iota = lax.iota(jnp.int32, 16)
    dqs = [iota + 16 * dq for dq in range(4)]

    def chunk_of(n):
        return n * nw + wid

    def fire_in(n, p):
        pltpu.async_copy(
            lutt_hbm.at[:, pl.ds(chunk_of(n) * VCHUNK, VCHUNK)],
            ins[p].at[:, pl.ds(0, VCHUNK)], isems[p])

    def wait_in(p):
        pltpu.make_async_copy(lutt_hbm.at[:, pl.ds(0, VCHUNK)],
                              ins[p].at[:, pl.ds(0, VCHUNK)],
                              isems[p]).wait()

    def xpose(src, dst, lo, hi, rbase):
        @pl.loop(lo, hi, step=8)
        def _(v0):
            vs = jnp.full((16,), v0, jnp.int32)
            for dv in range(8):
                v = v0 + dv
                vv = vs + dv
                r = lax.shift_right_logical(v, 1) - rbase
                c0 = lax.mul(lax.bitwise_and(v, 1), 64)
                for dq in range(4):
                    x = plsc.load_gather(src, [dqs[dq], vv])
                    dst[r, pl.ds(c0 + 16 * dq, 16)] = x

    def fire_out(n, p):
        pltpu.async_copy(ots[p],
                         out_hbm.at[pl.ds(chunk_of(n) * (VCHUNK // 2),
                                          VCHUNK // 2)],
                         osems[p])

    def wait_out(p):
        pltpu.make_async_copy(ots[p], out_hbm.at[pl.ds(0, VCHUNK // 2)],
                              osems[p]).wait()

    fire_in(0, 0)
    fire_in(1, 1)
    for n in range(2):
        wait_in(n)
        xpose(ins[n], ots[n], 0, VCHUNK, 0)
        fire_in(n + 2, n)
        fire_out(n, n)

    @pl.loop(1, nmain // 2 - 1)
    def _(go):
        for p in range(2):
            n = 2 * go + p
            wait_in(p)
            wait_out(p)
            xpose(ins[p], ots[p], 0, VCHUNK, 0)
            fire_in(n + 2, p)
            fire_out(n, p)

    for p in range(2):
        n = nmain - 2 + p
        wait_in(p)
        wait_out(p)
        xpose(ins[p], ots[p], 0, VCHUNK, 0)
        fire_out(n, p)
    wait_out(0)
    wait_out(1)

    @pl.when(wid < nextra)
    def _():
        t = nmain * nw + wid
        pltpu.sync_copy(lutt_hbm.at[:, pl.ds(t * VCHUNK, VCHUNK)],
                        in0.at[:, pl.ds(0, VCHUNK)])
        xpose(in0, ot0, 0, VCHUNK, 0)
        pltpu.sync_copy(ot0, out_hbm.at[pl.ds(t * (VCHUNK // 2),
                                              VCHUNK // 2)])

    if vocab % VCHUNK != 0:
        @pl.when(wid == nextra)
        def _():
            v0 = nchunks * VCHUNK
            pltpu.sync_copy(tail_hbm, tin)
            pltpu.sync_copy(tin, out_hbm.at[pl.ds(v0 // 2,
                                                  (vocab - v0) // 2)])


def _emb_body(idx_hbm, lut_hbm, out_hbm, idx_v, idx2_v, rows0, rows1,
              tb0, tb1, gsem0, gsem1, osem0, osem1, *, nseq):
    nc = plsc.get_sparse_core_info().num_cores
    wid = lax.axis_index("s") * nc + lax.axis_index("c")
    cps = BPW * BLK // CHUNK
    nchunks = nseq * cps
    rows = (rows0, rows1)
    tbs = (tb0, tb1)
    gsems = (gsem0, gsem1)
    osems = (osem0, osem1)

    iota = lax.iota(jnp.int32, 16)
    rowi2 = [[iota + 16 * j + 128 * u for j in range(8)] for u in range(2)]

    pltpu.sync_copy(idx_hbm.at[:, pl.ds(wid * BPW * BLK, BPW * BLK)], idx_v)

    def fire_gather(i, p):
        s = i // cps
        q = i % cps
        for u in range(2):
            for j in range(8):
                v = idx_v[s, pl.ds(q * CHUNK + 128 * u + 16 * j, 16)]
                idx2_v[p, u, pl.ds(16 * j, 16)] = (
                    lax.shift_right_logical(v, 1))
        for u in range(2):
            pltpu.async_copy(lut_hbm.at[idx2_v.at[p, u]],
                             rows[p].at[pl.ds(128 * u, BLK)], gsems[p])

    def wait_gather(p):
        for u in range(2):
            pltpu.make_async_copy(lut_hbm.at[idx2_v.at[p, 0]],
                                  rows[p].at[pl.ds(0, BLK)],
                                  gsems[p]).wait()

    def xpose_scale(i, p):
        s = i // cps
        q = i % cps
        for u in range(2):
            paroff = [
                lax.shift_left(
                    lax.bitwise_and(
                        idx_v[s, pl.ds(q * CHUNK + 128 * u + 16 * j, 16)],
                        1), 6)
                for j in range(8)
            ]

            @pl.loop(0, D_MODEL)
            def _(d0):
                dvec = lax.bitwise_and(
                    jnp.full((16,), d0, jnp.int32) + iota, D_MODEL - 1)
                for j in range(8):
                    c = dvec + paroff[j]
                    v = plsc.load_gather(rows[p], [rowi2[u][j], c])
                    plsc.store_scatter(tbs[p], [dvec, rowi2[u][j]],
                                       v * SCALE)

    def fire_store(i, p):
        s = i // cps
        b0 = wid * BPW * BLK + (i % cps) * CHUNK
        pltpu.async_copy(tbs[p], out_hbm.at[s].at[:, pl.ds(b0, CHUNK)],
                         osems[p])

    def wait_store(p):
        pltpu.make_async_copy(tbs[p], out_hbm.at[0].at[:, pl.ds(0, CHUNK)],
                              osems[p]).wait()

    fire_gather(0, 0)
    fire_gather(1, 1)
    for i in range(2):
        wait_gather(i)
        xpose_scale(i, i)
        fire_gather(i + 2, i)
        fire_store(i, i)

    @pl.loop(1, nchunks // 2 - 1)
    def _(go):
        for p in range(2):
            i = 2 * go + p
            wait_gather(p)
            wait_store(p)
            xpose_scale(i, p)
            fire_gather(i + 2, p)
            fire_store(i, p)

    for p in range(2):
        i = nchunks - 2 + p
        wait_gather(p)
        wait_store(p)
        xpose_scale(i, p)
        fire_store(i, p)
    wait_store(0)
    wait_store(1)


def kernel(input_data, lut):
    nbatch, nseq = input_data.shape
    vocab = lut.shape[0]
    nw = _nw()
    assert nbatch % (nw * BPW * BLK) == 0 and vocab % 2 == 0

    idx_t = input_data.T.astype(jnp.int32)
    lut_t = lut.T
    ntail = vocab % VCHUNK
    tail_rows = max(ntail // 2, 1)
    lut_tail = lut[vocab - max(ntail, 2):].reshape(tail_rows, 2 * D_MODEL)

    mesh = plsc.VectorSubcoreMesh(core_axis_name="c", subcore_axis_name="s")
    cparams = pltpu.CompilerParams(use_tc_tiling_on_sc=True,
                                   needs_layout_passes=False)

    repack = functools.partial(
        pl.kernel,
        mesh=mesh,
        out_type=jax.ShapeDtypeStruct((vocab // 2, 2 * D_MODEL),
                                      jnp.float32),
        scratch_types=[
            pltpu.VMEM((D_MODEL, SKEW), jnp.float32),
            pltpu.VMEM((D_MODEL, SKEW), jnp.float32),
            pltpu.VMEM((VCHUNK // 2, 2 * D_MODEL), jnp.float32),
            pltpu.VMEM((VCHUNK // 2, 2 * D_MODEL), jnp.float32),
            pltpu.VMEM((tail_rows, 2 * D_MODEL), jnp.float32),
            pltpu.SemaphoreType.DMA,
            pltpu.SemaphoreType.DMA,
            pltpu.SemaphoreType.DMA,
            pltpu.SemaphoreType.DMA,
        ],
        compiler_params=cparams,
    )(functools.partial(_repack_body, vocab=vocab))

    emb = functools.partial(
        pl.kernel,
        mesh=mesh,
        out_type=jax.ShapeDtypeStruct((nseq, D_MODEL, nbatch), jnp.float32),
        scratch_types=[
            pltpu.VMEM((nseq, BPW * BLK), jnp.int32),
            pltpu.VMEM((2, 2, BLK), jnp.int32),
            pltpu.VMEM((CHUNK, 2 * D_MODEL), jnp.float32),
            pltpu.VMEM((CHUNK, 2 * D_MODEL), jnp.float32),
            pltpu.VMEM((D_MODEL, CHUNK), jnp.float32),
            pltpu.VMEM((D_MODEL, CHUNK), jnp.float32),
            pltpu.SemaphoreType.DMA,
            pltpu.SemaphoreType.DMA,
            pltpu.SemaphoreType.DMA,
            pltpu.SemaphoreType.DMA,
        ],
        compiler_params=cparams,
    )(functools.partial(_emb_body, nseq=nseq))

    lut_packed = repack(lut_t, lut_tail)
    out = emb(idx_t, lut_packed)
    return jnp.transpose(out, (2, 0, 1))


# speedup vs baseline: 1.9801x; 1.9801x over previous
import functools
import math

import jax
import jax.numpy as jnp
from jax import lax
from jax.experimental import pallas as pl
from jax.experimental.pallas import tpu as pltpu
from jax.experimental.pallas import tpu_sc as plsc

D_MODEL = 64
SCALE = math.sqrt(D_MODEL)
BLK = 128
CHUNK = 256
BPW = 4
VCHUNK = 128


def _nw():
    info = plsc.get_sparse_core_info()
    return info.num_cores * info.num_subcores


def _repack_body(lutt_hbm, tail_hbm, out_hbm, in0, in1, ot0, ot1, tin,
                 isem0, isem1, osem0, osem1, *, vocab):
    nc = plsc.get_sparse_core_info().num_cores
    nw = _nw()
    wid = lax.axis_index("s") * nc + lax.axis_index("c")
    nchunks = vocab // VCHUNK
    nmain = nchunks // nw
    nextra = nchunks - nmain * nw
    ins = (in0, in1)
    ots = (ot0, ot1)
    isems = (isem0, isem1)
    osems = (osem0, osem1)

    iota = lax.iota(jnp.int32, 16)
    iotah = lax.shift_right_logical(iota, 1)
    par64 = lax.shift_left(lax.bitwise_and(iota, 1), 6)
    vvs = [iota + 16 * k for k in range(8)]
    rrs = [iotah + 8 * k for k in range(8)]

    def chunk_of(n):
        return n * nw + wid

    def fire_in(n, p):
        pltpu.async_copy(
            lutt_hbm.at[:, pl.ds(chunk_of(n) * VCHUNK, VCHUNK)],
            ins[p], isems[p])

    def wait_in(p):
        pltpu.make_async_copy(lutt_hbm.at[:, pl.ds(0, VCHUNK)], ins[p],
                              isems[p]).wait()

    def xpose(src, dst):
        @pl.loop(0, 16)
        def _(r):
            rot = lax.bitwise_and(iota + r, 15)
            for dq in range(4):
                dvec = rot + (16 * dq)
                cvec = dvec + par64
                for k in range(8):
                    v = plsc.load_gather(src, [dvec, vvs[k]])
                    plsc.store_scatter(dst, [rrs[k], cvec], v)

    def fire_out(n, p):
        pltpu.async_copy(ots[p],
                         out_hbm.at[pl.ds(chunk_of(n) * (VCHUNK // 2),
                                          VCHUNK // 2)],
                         osems[p])

    def wait_out(p):
        pltpu.make_async_copy(ots[p], out_hbm.at[pl.ds(0, VCHUNK // 2)],
                              osems[p]).wait()

    fire_in(0, 0)
    fire_in(1, 1)
    for n in range(2):
        wait_in(n)
        xpose(ins[n], ots[n])
        fire_in(n + 2, n)
        fire_out(n, n)

    @pl.loop(1, nmain // 2 - 1)
    def _(go):
        for p in range(2):
            n = 2 * go + p
            wait_in(p)
            wait_out(p)
            xpose(ins[p], ots[p])
            fire_in(n + 2, p)
            fire_out(n, p)

    for p in range(2):
        n = nmain - 2 + p
        wait_in(p)
        wait_out(p)
        xpose(ins[p], ots[p])
        fire_out(n, p)
    wait_out(0)
    wait_out(1)

    @pl.when(wid < nextra)
    def _():
        t = nmain * nw + wid
        pltpu.sync_copy(lutt_hbm.at[:, pl.ds(t * VCHUNK, VCHUNK)], in0)
        xpose(in0, ot0)
        pltpu.sync_copy(ot0, out_hbm.at[pl.ds(t * (VCHUNK // 2),
                                              VCHUNK // 2)])

    if vocab % VCHUNK != 0:
        @pl.when(wid == nextra)
        def _():
            v0 = nchunks * VCHUNK
            pltpu.sync_copy(tail_hbm, tin)
            pltpu.sync_copy(tin, out_hbm.at[pl.ds(v0 // 2,
                                                  (vocab - v0) // 2)])


def _emb_body(idx_hbm, lut_hbm, out_hbm, idx_v, idx2_v, rows0, rows1,
              tb0, tb1, gsem0, gsem1, osem0, osem1, *, nseq):
    nc = plsc.get_sparse_core_info().num_cores
    wid = lax.axis_index("s") * nc + lax.axis_index("c")
    cps = BPW * BLK // CHUNK
    nchunks = nseq * cps
    rows = (rows0, rows1)
    tbs = (tb0, tb1)
    gsems = (gsem0, gsem1)
    osems = (osem0, osem1)

    iota = lax.iota(jnp.int32, 16)
    rowi2 = [[iota + 16 * j + 128 * u for j in range(8)] for u in range(2)]

    pltpu.sync_copy(idx_hbm.at[:, pl.ds(wid * BPW * BLK, BPW * BLK)], idx_v)

    def fire_gather(i, p):
        s = i // cps
        q = i % cps
        for u in range(2):
            for j in range(8):
                v = idx_v[s, pl.ds(q * CHUNK + 128 * u + 16 * j, 16)]
                idx2_v[p, u, pl.ds(16 * j, 16)] = (
                    lax.shift_right_logical(v, 1))
        for u in range(2):
            pltpu.async_copy(lut_hbm.at[idx2_v.at[p, u]],
                             rows[p].at[pl.ds(128 * u, BLK)], gsems[p])

    def wait_gather(p):
        for u in range(2):
            pltpu.make_async_copy(lut_hbm.at[idx2_v.at[p, 0]],
                                  rows[p].at[pl.ds(0, BLK)],
                                  gsems[p]).wait()

    def xpose_scale(i, p):
        s = i // cps
        q = i % cps
        for u in range(2):
            paroff = [
                lax.shift_left(
                    lax.bitwise_and(
                        idx_v[s, pl.ds(q * CHUNK + 128 * u + 16 * j, 16)],
                        1), 6)
                for j in range(8)
            ]

            @pl.loop(0, D_MODEL, unroll=2)
            def _(d0):
                dvec = lax.bitwise_and(
                    jnp.full((16,), d0, jnp.int32) + iota, D_MODEL - 1)
                for j in range(8):
                    c = dvec + paroff[j]
                    v = plsc.load_gather(rows[p], [rowi2[u][j], c])
                    plsc.store_scatter(tbs[p], [dvec, rowi2[u][j]],
                                       v * SCALE)

    def fire_store(i, p):
        s = i // cps
        b0 = wid * BPW * BLK + (i % cps) * CHUNK
        pltpu.async_copy(tbs[p], out_hbm.at[s].at[:, pl.ds(b0, CHUNK)],
                         osems[p])

    def wait_store(p):
        pltpu.make_async_copy(tbs[p], out_hbm.at[0].at[:, pl.ds(0, CHUNK)],
                              osems[p]).wait()

    fire_gather(0, 0)
    fire_gather(1, 1)
    for i in range(2):
        wait_gather(i)
        xpose_scale(i, i)
        fire_gather(i + 2, i)
        fire_store(i, i)

    @pl.loop(1, nchunks // 2 - 1)
    def _(go):
        for p in range(2):
            i = 2 * go + p
            wait_gather(p)
            wait_store(p)
            xpose_scale(i, p)
            fire_gather(i + 2, p)
            fire_store(i, p)

    for p in range(2):
        i = nchunks - 2 + p
        wait_gather(p)
        wait_store(p)
        xpose_scale(i, p)
        fire_store(i, p)
    wait_store(0)
    wait_store(1)


def kernel(input_data, lut):
    nbatch, nseq = input_data.shape
    vocab = lut.shape[0]
    nw = _nw()
    assert nbatch % (nw * BPW * BLK) == 0 and vocab % 2 == 0

    idx_t = input_data.T.astype(jnp.int32)
    lut_t = lut.T
    ntail = vocab % VCHUNK
    tail_rows = max(ntail // 2, 1)
    lut_tail = lut[vocab - max(ntail, 2):].reshape(tail_rows, 2 * D_MODEL)

    mesh = plsc.VectorSubcoreMesh(core_axis_name="c", subcore_axis_name="s")
    cparams = pltpu.CompilerParams(use_tc_tiling_on_sc=True,
                                   needs_layout_passes=False)

    repack = functools.partial(
        pl.kernel,
        mesh=mesh,
        out_type=jax.ShapeDtypeStruct((vocab // 2, 2 * D_MODEL),
                                      jnp.float32),
        scratch_types=[
            pltpu.VMEM((D_MODEL, VCHUNK), jnp.float32),
            pltpu.VMEM((D_MODEL, VCHUNK), jnp.float32),
            pltpu.VMEM((VCHUNK // 2, 2 * D_MODEL), jnp.float32),
            pltpu.VMEM((VCHUNK // 2, 2 * D_MODEL), jnp.float32),
            pltpu.VMEM((tail_rows, 2 * D_MODEL), jnp.float32),
            pltpu.SemaphoreType.DMA,
            pltpu.SemaphoreType.DMA,
            pltpu.SemaphoreType.DMA,
            pltpu.SemaphoreType.DMA,
        ],
        compiler_params=cparams,
    )(functools.partial(_repack_body, vocab=vocab))

    emb = functools.partial(
        pl.kernel,
        mesh=mesh,
        out_type=jax.ShapeDtypeStruct((nseq, D_MODEL, nbatch), jnp.float32),
        scratch_types=[
            pltpu.VMEM((nseq, BPW * BLK), jnp.int32),
            pltpu.VMEM((2, 2, BLK), jnp.int32),
            pltpu.VMEM((CHUNK, 2 * D_MODEL), jnp.float32),
            pltpu.VMEM((CHUNK, 2 * D_MODEL), jnp.float32),
            pltpu.VMEM((D_MODEL, CHUNK), jnp.float32),
            pltpu.VMEM((D_MODEL, CHUNK), jnp.float32),
            pltpu.SemaphoreType.DMA,
            pltpu.SemaphoreType.DMA,
            pltpu.SemaphoreType.DMA,
            pltpu.SemaphoreType.DMA,
        ],
        compiler_params=cparams,
    )(functools.partial(_emb_body, nseq=nseq))

    lut_packed = repack(lut_t, lut_tail)
    out = emb(idx_t, lut_packed)
    return jnp.transpose(out, (2, 0, 1))
